# baseline (speedup 1.0000x reference)
import jax
import jax.numpy as jnp
from jax import lax
from jax.experimental import pallas as pl
from jax.experimental.pallas import tpu as pltpu, tpu_sc as plsc

N_NODES = 10000
D_IN = 256
HEADS = 8
D_HEAD = 64
N_CLASSES = 40
NEG = 0.2

N_PAD = 10240
ROWS_T = N_PAD // 16
E_TOT = N_NODES + 160000
E_PAD = 172032
CHUNK = 128
CT1 = E_PAD // 16
NCH1 = CT1 // CHUNK
CT2 = E_PAD // 32
NCH2 = CT2 // CHUNK
NBUF1 = 4
NBUF2 = 3
F2 = 48
TILE_N = 256
GRID_N = N_PAD // TILE_N
EU = 8
ZR = 64

_mesh = plsc.VectorSubcoreMesh(core_axis_name="c", subcore_axis_name="s")


def _lrelu(x):
    return jnp.maximum(x, NEG * x)


def _k1_body(x_ref, w_ref, a_ref, h_ref, asd_ref):
    h = jnp.dot(x_ref[...], w_ref[...], preferred_element_type=jnp.float32)
    h_ref[...] = h
    asd_ref[...] = jnp.dot(h, a_ref[...], preferred_element_type=jnp.float32)


_k1 = pl.pallas_call(
    _k1_body,
    grid=(GRID_N,),
    in_specs=[
        pl.BlockSpec((TILE_N, D_IN), lambda i: (i, 0)),
        pl.BlockSpec((D_IN, HEADS * D_HEAD), lambda i: (0, 0)),
        pl.BlockSpec((HEADS * D_HEAD, 16), lambda i: (0, 0)),
    ],
    out_specs=[
        pl.BlockSpec((TILE_N, HEADS * D_HEAD), lambda i: (i, 0)),
        pl.BlockSpec((TILE_N, 16), lambda i: (i, 0)),
    ],
    out_shape=[
        jax.ShapeDtypeStruct((N_PAD, HEADS * D_HEAD), jnp.float32),
        jax.ShapeDtypeStruct((N_PAD, 16), jnp.float32),
    ],
)


def _k2_body(num_ref, den_ref, b1_ref, w2_ref, a2_ref, g_ref, asd2_ref):
    parts = []
    for hh in range(HEADS):
        t = num_ref[hh] / den_ref[:, hh:hh + 1]
        t = t + b1_ref[0, hh * D_HEAD:(hh + 1) * D_HEAD]
        parts.append(t)
    t = jnp.concatenate(parts, axis=1)
    t = jnp.where(t > 0, t, jnp.exp(t) - 1.0)
    g = jnp.dot(t, w2_ref[...], preferred_element_type=jnp.float32)
    g_ref[...] = g
    asd2_ref[...] = jnp.dot(g, a2_ref[...], preferred_element_type=jnp.float32)


_k2 = pl.pallas_call(
    _k2_body,
    grid=(GRID_N,),
    in_specs=[
        pl.BlockSpec((HEADS, TILE_N, D_HEAD), lambda i: (0, i, 0)),
        pl.BlockSpec((TILE_N, HEADS), lambda i: (i, 0)),
        pl.BlockSpec((1, HEADS * D_HEAD), lambda i: (0, 0)),
        pl.BlockSpec((HEADS * D_HEAD, F2), lambda i: (0, 0)),
        pl.BlockSpec((F2, 16), lambda i: (0, 0)),
    ],
    out_specs=[
        pl.BlockSpec((TILE_N, F2), lambda i: (i, 0)),
        pl.BlockSpec((TILE_N, 16), lambda i: (i, 0)),
    ],
    out_shape=[
        jax.ShapeDtypeStruct((N_PAD, F2), jnp.float32),
        jax.ShapeDtypeStruct((N_PAD, 16), jnp.float32),
    ],
)


def _k3_body(num_ref, den_ref, b2_ref, out_ref):
    n = num_ref[0] + num_ref[1]
    dd = den_ref[:, 0:1] + den_ref[:, 1:2]
    out_ref[...] = n / dd + b2_ref[0]


_k3 = pl.pallas_call(
    _k3_body,
    grid=(GRID_N,),
    in_specs=[
        pl.BlockSpec((2, TILE_N, F2), lambda i: (0, i, 0)),
        pl.BlockSpec((TILE_N, 2), lambda i: (i, 0)),
        pl.BlockSpec((1, F2), lambda i: (0, 0)),
    ],
    out_specs=pl.BlockSpec((TILE_N, F2), lambda i: (i, 0)),
    out_shape=jax.ShapeDtypeStruct((N_PAD, F2), jnp.float32),
)


_sc_params = pltpu.CompilerParams(needs_layout_passes=False,
                                  use_tc_tiling_on_sc=False)


def _zero_ref(ref, n16):
    def _z(i, _):
        ref[pl.ds(i * 16, 16)] = jnp.zeros((16,), jnp.float32)
        return 0
    lax.fori_loop(0, n16, _z, 0)


def _zero_rows(ref, nrows, ncol16):
    def _z(i, _):
        for q in range(ncol16):
            ref[i, pl.ds(q * 16, 16)] = jnp.zeros((16,), jnp.float32)
        return 0
    lax.fori_loop(0, nrows, _z, 0)


def _sc1_body(feat, asdt, srcp, dstp,
              num1, den1,
              asv, adv, srcv, dstv,
              idxf, idxa, wb, rows, zrows, zden, dvbuf,
              sg, ss, sd,
              accs, dens):
    c = lax.axis_index("c")
    s = lax.axis_index("s")

    pltpu.sync_copy(srcp.at[pl.ds(s * CT1, CT1)], srcv)
    pltpu.sync_copy(dstp.at[pl.ds(s * CT1, CT1)], dstv)

    _zero_rows(zrows, ZR, D_HEAD // 16)
    _zero_ref(zden, ROWS_T // 16)
    pltpu.sync_copy(zden, dens.at[pl.ds(s * ROWS_T, ROWS_T)])

    for i in range(HEADS // 2):
        hg = c * (HEADS // 2) + i

        pltpu.sync_copy(asdt.at[hg], asv)
        pltpu.sync_copy(asdt.at[HEADS + hg], adv)
        for r in range(ROWS_T // ZR):
            pltpu.sync_copy(zrows, accs.at[pl.ds(s * ROWS_T + r * ZR, ZR)])
        plsc.subcore_barrier()

        def _main(j, _):
            for b in range(NBUF1):
                k = j * NBUF1 + b

                @pl.when(j > 0)
                def _w():
                    pltpu.make_async_copy(rows[b], accs.at[idxa[b]],
                                          ss[b]).wait()
                    pltpu.make_async_copy(wb[b], dens.at[idxa[b]],
                                          sd[b]).wait()
                for g in range(CHUNK // 16):
                    co = g * 16
                    sl = pl.ds(k * CHUNK + co, 16)
                    s16 = srcv[sl]
                    d16 = dstv[sl]
                    a16 = plsc.load_gather(asv, [s16])
                    b16 = plsc.load_gather(adv, [d16])
                    wb[b][pl.ds(co, 16)] = jnp.exp(_lrelu(a16 + b16))
                    idxf[b][pl.ds(co, 16)] = s16 * HEADS + hg
                    idxa[b][pl.ds(co, 16)] = d16
                pltpu.async_copy(feat.at[idxf[b]], rows[b], sg[b])
            for b in range(NBUF1):
                pltpu.async_copy(wb[b], dens.at[idxa[b]], sd[b], add=True)
                pltpu.make_async_copy(feat.at[idxf[b]], rows[b],
                                      sg[b]).wait()

                def _scale(e, _):
                    for u in range(EU):
                        ei = e * EU + u
                        wv = plsc.load_gather(
                            wb[b], [jnp.full((16,), ei, jnp.int32)])
                        for q in range(D_HEAD // 16):
                            rows[b][ei, pl.ds(q * 16, 16)] = (
                                rows[b][ei, pl.ds(q * 16, 16)] * wv)
                    return 0
                lax.fori_loop(0, CHUNK // EU, _scale, 0)

                pltpu.async_copy(rows[b], accs.at[idxa[b]], ss[b],
                                 add=True)
            return 0
        lax.fori_loop(0, NCH1 // NBUF1, _main, 0)

        for b in range(NBUF1):
            pltpu.make_async_copy(rows[b], accs.at[idxa[b]], ss[b]).wait()
            pltpu.make_async_copy(wb[b], dens.at[idxa[b]], sd[b]).wait()
        plsc.subcore_barrier()

        for r in range(ROWS_T // CHUNK):
            off = s * ROWS_T + r * CHUNK
            pltpu.sync_copy(accs.at[pl.ds(off, CHUNK)], rows[0])
            pltpu.sync_copy(rows[0], num1.at[hg, pl.ds(off, CHUNK)])
        pltpu.sync_copy(dens.at[pl.ds(s * ROWS_T, ROWS_T)], dvbuf)
        pltpu.sync_copy(dvbuf, den1.at[hg, pl.ds(s * ROWS_T, ROWS_T)])
        if i < HEADS // 2 - 1:
            pltpu.sync_copy(zden, dens.at[pl.ds(s * ROWS_T, ROWS_T)])
        plsc.subcore_barrier()


_sc1 = pl.kernel(
    _sc1_body,
    compiler_params=_sc_params,
    out_type=[
        jax.ShapeDtypeStruct((HEADS, N_PAD, D_HEAD), jnp.float32),
        jax.ShapeDtypeStruct((HEADS, N_PAD), jnp.float32),
    ],
    mesh=_mesh,
    scratch_types=[
        pltpu.VMEM((N_PAD,), jnp.float32),
        pltpu.VMEM((N_PAD,), jnp.float32),
        pltpu.VMEM((CT1,), jnp.int32),
        pltpu.VMEM((CT1,), jnp.int32),
        [pltpu.VMEM((CHUNK,), jnp.int32)] * NBUF1,
        [pltpu.VMEM((CHUNK,), jnp.int32)] * NBUF1,
        [pltpu.VMEM((CHUNK,), jnp.float32)] * NBUF1,
        [pltpu.VMEM((CHUNK, D_HEAD), jnp.float32)] * NBUF1,
        pltpu.VMEM((ZR, D_HEAD), jnp.float32),
        pltpu.VMEM((ROWS_T,), jnp.float32),
        pltpu.VMEM((ROWS_T,), jnp.float32),
        [pltpu.SemaphoreType.DMA] * NBUF1,
        [pltpu.SemaphoreType.DMA] * NBUF1,
        [pltpu.SemaphoreType.DMA] * NBUF1,
        pltpu.VMEM_SHARED((N_PAD, D_HEAD), jnp.float32),
        pltpu.VMEM_SHARED((N_PAD,), jnp.float32),
    ],
)


def _sc2_body(g, asdt2, srcp, dstp,
              num2, den2,
              asv, adv, srcv, dstv,
              idxs, idxa, wb, rows, zrows, zden, dvbuf,
              sg, ss, sd,
              accs, dens):
    c = lax.axis_index("c")
    s = lax.axis_index("s")

    base0 = c * (16 * CT2) + s * CT2
    pltpu.sync_copy(srcp.at[pl.ds(base0, CT2)], srcv)
    pltpu.sync_copy(dstp.at[pl.ds(base0, CT2)], dstv)
    pltpu.sync_copy(asdt2.at[0], asv)
    pltpu.sync_copy(asdt2.at[1], adv)

    _zero_rows(zrows, ZR, F2 // 16)
    _zero_ref(zden, ROWS_T // 16)
    pltpu.sync_copy(zden, dens.at[pl.ds(s * ROWS_T, ROWS_T)])
    for r in range(ROWS_T // ZR):
        pltpu.sync_copy(zrows, accs.at[pl.ds(s * ROWS_T + r * ZR, ZR)])
    plsc.subcore_barrier()

    def _main(j, _):
        for b in range(NBUF2):
            k = j * NBUF2 + b

            @pl.when(j > 0)
            def _w():
                pltpu.make_async_copy(rows[b], accs.at[idxa[b]],
                                      ss[b]).wait()
                pltpu.make_async_copy(wb[b], dens.at[idxa[b]],
                                      sd[b]).wait()
            for gi in range(CHUNK // 16):
                co = gi * 16
                sl = pl.ds(k * CHUNK + co, 16)
                s16 = srcv[sl]
                d16 = dstv[sl]
                a16 = plsc.load_gather(asv, [s16])
                b16 = plsc.load_gather(adv, [d16])
                wb[b][pl.ds(co, 16)] = jnp.exp(_lrelu(a16 + b16))
                idxs[b][pl.ds(co, 16)] = s16
                idxa[b][pl.ds(co, 16)] = d16
            pltpu.async_copy(g.at[idxs[b]], rows[b], sg[b])
        for b in range(NBUF2):
            pltpu.async_copy(wb[b], dens.at[idxa[b]], sd[b], add=True)
            pltpu.make_async_copy(g.at[idxs[b]], rows[b], sg[b]).wait()

            def _scale(e, _):
                for u in range(EU):
                    ei = e * EU + u
                    wv = plsc.load_gather(
                        wb[b], [jnp.full((16,), ei, jnp.int32)])
                    for q in range(F2 // 16):
                        rows[b][ei, pl.ds(q * 16, 16)] = (
                            rows[b][ei, pl.ds(q * 16, 16)] * wv)
                return 0
            lax.fori_loop(0, CHUNK // EU, _scale, 0)

            pltpu.async_copy(rows[b], accs.at[idxa[b]], ss[b], add=True)
        return 0
    lax.fori_loop(0, NCH2 // NBUF2, _main, 0)

    for b in range(NBUF2):
        pltpu.make_async_copy(rows[b], accs.at[idxa[b]], ss[b]).wait()
        pltpu.make_async_copy(wb[b], dens.at[idxa[b]], sd[b]).wait()
    plsc.subcore_barrier()

    for r in range(ROWS_T // CHUNK):
        off = s * ROWS_T + r * CHUNK
        pltpu.sync_copy(accs.at[pl.ds(off, CHUNK)], rows[0])
        pltpu.sync_copy(rows[0], num2.at[c, pl.ds(off, CHUNK)])
    pltpu.sync_copy(dens.at[pl.ds(s * ROWS_T, ROWS_T)], dvbuf)
    pltpu.sync_copy(dvbuf, den2.at[c, pl.ds(s * ROWS_T, ROWS_T)])


_sc2 = pl.kernel(
    _sc2_body,
    compiler_params=_sc_params,
    out_type=[
        jax.ShapeDtypeStruct((2, N_PAD, F2), jnp.float32),
        jax.ShapeDtypeStruct((2, N_PAD), jnp.float32),
    ],
    mesh=_mesh,
    scratch_types=[
        pltpu.VMEM((N_PAD,), jnp.float32),
        pltpu.VMEM((N_PAD,), jnp.float32),
        pltpu.VMEM((CT2,), jnp.int32),
        pltpu.VMEM((CT2,), jnp.int32),
        [pltpu.VMEM((CHUNK,), jnp.int32)] * NBUF2,
        [pltpu.VMEM((CHUNK,), jnp.int32)] * NBUF2,
        [pltpu.VMEM((CHUNK,), jnp.float32)] * NBUF2,
        [pltpu.VMEM((CHUNK, F2), jnp.float32)] * NBUF2,
        pltpu.VMEM((ZR, F2), jnp.float32),
        pltpu.VMEM((ROWS_T,), jnp.float32),
        pltpu.VMEM((ROWS_T,), jnp.float32),
        [pltpu.SemaphoreType.DMA] * NBUF2,
        [pltpu.SemaphoreType.DMA] * NBUF2,
        [pltpu.SemaphoreType.DMA] * NBUF2,
        pltpu.VMEM_SHARED((N_PAD, F2), jnp.float32),
        pltpu.VMEM_SHARED((N_PAD,), jnp.float32),
    ],
)


def kernel(x, edge_index, W1, a_src1, a_dst1, b1, W2, a_src2, a_dst2, b2):
    loops = jnp.arange(N_NODES, dtype=edge_index.dtype)
    src = jnp.concatenate([edge_index[0], loops]).astype(jnp.int32)
    dst = jnp.concatenate([edge_index[1], loops]).astype(jnp.int32)
    srcp = jnp.concatenate(
        [src, jnp.zeros((E_PAD - E_TOT,), jnp.int32)])
    dstp = jnp.concatenate(
        [dst, jnp.full((E_PAD - E_TOT,), N_NODES, jnp.int32)])

    xp = jnp.concatenate(
        [x, jnp.zeros((N_PAD - N_NODES, D_IN), jnp.float32)], axis=0)

    eye = jnp.eye(HEADS, dtype=jnp.float32)
    A_s = (a_src1[:, :, None] * eye[:, None, :]).reshape(HEADS * D_HEAD, HEADS)
    A_d = (a_dst1[:, :, None] * eye[:, None, :]).reshape(HEADS * D_HEAD, HEADS)
    A1 = jnp.concatenate([A_s, A_d], axis=1)

    h, asd = _k1(xp, W1, A1)
    feat = h.reshape(N_PAD * HEADS, D_HEAD)
    asdt = asd.T

    num1, den1 = _sc1(feat, asdt, srcp, dstp)

    W2p = jnp.concatenate(
        [W2, jnp.zeros((HEADS * D_HEAD, F2 - N_CLASSES), jnp.float32)], axis=1)
    a2 = jnp.zeros((F2, 16), jnp.float32)
    a2 = a2.at[:N_CLASSES, 0].set(a_src2[0])
    a2 = a2.at[:N_CLASSES, 1].set(a_dst2[0])

    g, asd2 = _k2(num1, den1.T, b1.reshape(1, -1), W2p, a2)
    asdt2 = asd2.T

    num2, den2 = _sc2(g, asdt2, srcp, dstp)

    b2p = jnp.concatenate(
        [b2, jnp.zeros((F2 - N_CLASSES,), jnp.float32)]).reshape(1, F2)
    outp = _k3(num2, den2.T, b2p)
    return outp[:N_NODES, :N_CLASSES]


# speedup vs baseline: 1.0109x; 1.0109x over previous
import jax
import jax.numpy as jnp
from jax import lax
from jax.experimental import pallas as pl
from jax.experimental.pallas import tpu as pltpu, tpu_sc as plsc

N_NODES = 10000
D_IN = 256
HEADS = 8
D_HEAD = 64
N_CLASSES = 40
NEG = 0.2

N_PAD = 10240
ROWS_T = N_PAD // 16
E_TOT = N_NODES + 160000
E_PAD = 172032
CHUNK = 128
CT1 = E_PAD // 16
NCH1 = CT1 // CHUNK
CT2 = E_PAD // 32
NCH2 = CT2 // CHUNK
NBUF1 = 4
NBUF2 = 3
F2 = 48
TILE_N = 256
GRID_N = N_PAD // TILE_N
EU = 4
ZR = 64

_mesh = plsc.VectorSubcoreMesh(core_axis_name="c", subcore_axis_name="s")


def _lrelu(x):
    return jnp.maximum(x, NEG * x)


def _k1_body(x_ref, w_ref, a_ref, h_ref, asd_ref):
    h = jnp.dot(x_ref[...], w_ref[...], preferred_element_type=jnp.float32)
    h_ref[...] = h
    asd_ref[...] = jnp.dot(h, a_ref[...], preferred_element_type=jnp.float32)


_k1 = pl.pallas_call(
    _k1_body,
    grid=(GRID_N,),
    in_specs=[
        pl.BlockSpec((TILE_N, D_IN), lambda i: (i, 0)),
        pl.BlockSpec((D_IN, HEADS * D_HEAD), lambda i: (0, 0)),
        pl.BlockSpec((HEADS * D_HEAD, 16), lambda i: (0, 0)),
    ],
    out_specs=[
        pl.BlockSpec((TILE_N, HEADS * D_HEAD), lambda i: (i, 0)),
        pl.BlockSpec((TILE_N, 16), lambda i: (i, 0)),
    ],
    out_shape=[
        jax.ShapeDtypeStruct((N_PAD, HEADS * D_HEAD), jnp.float32),
        jax.ShapeDtypeStruct((N_PAD, 16), jnp.float32),
    ],
)


def _k2_body(num_ref, den_ref, b1_ref, w2_ref, a2_ref, g_ref, asd2_ref):
    parts = []
    for hh in range(HEADS):
        t = num_ref[hh] / den_ref[:, hh:hh + 1]
        t = t + b1_ref[0, hh * D_HEAD:(hh + 1) * D_HEAD]
        parts.append(t)
    t = jnp.concatenate(parts, axis=1)
    t = jnp.where(t > 0, t, jnp.exp(t) - 1.0)
    g = jnp.dot(t, w2_ref[...], preferred_element_type=jnp.float32)
    g_ref[...] = g
    asd2_ref[...] = jnp.dot(g, a2_ref[...], preferred_element_type=jnp.float32)


_k2 = pl.pallas_call(
    _k2_body,
    grid=(GRID_N,),
    in_specs=[
        pl.BlockSpec((HEADS, TILE_N, D_HEAD), lambda i: (0, i, 0)),
        pl.BlockSpec((TILE_N, HEADS), lambda i: (i, 0)),
        pl.BlockSpec((1, HEADS * D_HEAD), lambda i: (0, 0)),
        pl.BlockSpec((HEADS * D_HEAD, F2), lambda i: (0, 0)),
        pl.BlockSpec((F2, 16), lambda i: (0, 0)),
    ],
    out_specs=[
        pl.BlockSpec((TILE_N, F2), lambda i: (i, 0)),
        pl.BlockSpec((TILE_N, 16), lambda i: (i, 0)),
    ],
    out_shape=[
        jax.ShapeDtypeStruct((N_PAD, F2), jnp.float32),
        jax.ShapeDtypeStruct((N_PAD, 16), jnp.float32),
    ],
)


def _k3_body(num_ref, den_ref, b2_ref, out_ref):
    n = num_ref[0] + num_ref[1]
    dd = den_ref[:, 0:1] + den_ref[:, 1:2]
    out_ref[...] = n / dd + b2_ref[0]


_k3 = pl.pallas_call(
    _k3_body,
    grid=(GRID_N,),
    in_specs=[
        pl.BlockSpec((2, TILE_N, F2), lambda i: (0, i, 0)),
        pl.BlockSpec((TILE_N, 2), lambda i: (i, 0)),
        pl.BlockSpec((1, F2), lambda i: (0, 0)),
    ],
    out_specs=pl.BlockSpec((TILE_N, F2), lambda i: (i, 0)),
    out_shape=jax.ShapeDtypeStruct((N_PAD, F2), jnp.float32),
)


_sc_params = pltpu.CompilerParams(needs_layout_passes=False,
                                  use_tc_tiling_on_sc=False)


def _zero_ref(ref, n16):
    def _z(i, _):
        ref[pl.ds(i * 16, 16)] = jnp.zeros((16,), jnp.float32)
        return 0
    lax.fori_loop(0, n16, _z, 0)


def _zero_rows(ref, nrows, ncol16):
    def _z(i, _):
        for q in range(ncol16):
            ref[i, pl.ds(q * 16, 16)] = jnp.zeros((16,), jnp.float32)
        return 0
    lax.fori_loop(0, nrows, _z, 0)


def _sc1_body(feat, asdt, srcp, dstp,
              num1, den1,
              asv, adv, srcv, dstv,
              idxf, idxa, wb, rows, zrows, zden, dvbuf,
              sg, ss, sd,
              accs, dens):
    c = lax.axis_index("c")
    s = lax.axis_index("s")

    pltpu.sync_copy(srcp.at[pl.ds(s * CT1, CT1)], srcv)
    pltpu.sync_copy(dstp.at[pl.ds(s * CT1, CT1)], dstv)

    _zero_rows(zrows, ZR, D_HEAD // 16)
    _zero_ref(zden, ROWS_T // 16)
    pltpu.sync_copy(zden, dens.at[pl.ds(s * ROWS_T, ROWS_T)])

    for i in range(HEADS // 2):
        hg = c * (HEADS // 2) + i

        pltpu.sync_copy(asdt.at[hg], asv)
        pltpu.sync_copy(asdt.at[HEADS + hg], adv)
        for r in range(ROWS_T // ZR):
            pltpu.sync_copy(zrows, accs.at[pl.ds(s * ROWS_T + r * ZR, ZR)])
        plsc.subcore_barrier()

        def _main(j, _):
            for b in range(NBUF1):
                k = j * NBUF1 + b

                @pl.when(j > 0)
                def _w():
                    pltpu.make_async_copy(rows[b], accs.at[idxa[b]],
                                          ss[b]).wait()
                    pltpu.make_async_copy(wb[b], dens.at[idxa[b]],
                                          sd[b]).wait()
                for g in range(CHUNK // 16):
                    co = g * 16
                    sl = pl.ds(k * CHUNK + co, 16)
                    s16 = srcv[sl]
                    d16 = dstv[sl]
                    a16 = plsc.load_gather(asv, [s16])
                    b16 = plsc.load_gather(adv, [d16])
                    wb[b][pl.ds(co, 16)] = jnp.exp(_lrelu(a16 + b16))
                    idxf[b][pl.ds(co, 16)] = s16 * HEADS + hg
                    idxa[b][pl.ds(co, 16)] = d16
                pltpu.async_copy(feat.at[idxf[b]], rows[b], sg[b])
            for b in range(NBUF1):
                pltpu.make_async_copy(feat.at[idxf[b]], rows[b],
                                      sg[b]).wait()

                def _scale(e, _):
                    for u in range(EU):
                        ei = e * EU + u
                        wv = plsc.load_gather(
                            wb[b], [jnp.full((16,), ei, jnp.int32)])
                        for q in range(D_HEAD // 16):
                            rows[b][ei, pl.ds(q * 16, 16)] = (
                                rows[b][ei, pl.ds(q * 16, 16)] * wv)
                    return 0
                lax.fori_loop(0, CHUNK // EU, _scale, 0)

                pltpu.async_copy(rows[b], accs.at[idxa[b]], ss[b],
                                 add=True)
                pltpu.async_copy(wb[b], dens.at[idxa[b]], sd[b], add=True)
            return 0
        lax.fori_loop(0, NCH1 // NBUF1, _main, 0)

        for b in range(NBUF1):
            pltpu.make_async_copy(rows[b], accs.at[idxa[b]], ss[b]).wait()
            pltpu.make_async_copy(wb[b], dens.at[idxa[b]], sd[b]).wait()
        plsc.subcore_barrier()

        for r in range(ROWS_T // CHUNK):
            off = s * ROWS_T + r * CHUNK
            pltpu.sync_copy(accs.at[pl.ds(off, CHUNK)], rows[0])
            pltpu.sync_copy(rows[0], num1.at[hg, pl.ds(off, CHUNK)])
        pltpu.sync_copy(dens.at[pl.ds(s * ROWS_T, ROWS_T)], dvbuf)
        pltpu.sync_copy(dvbuf, den1.at[hg, pl.ds(s * ROWS_T, ROWS_T)])
        if i < HEADS // 2 - 1:
            pltpu.sync_copy(zden, dens.at[pl.ds(s * ROWS_T, ROWS_T)])
        plsc.subcore_barrier()


_sc1 = pl.kernel(
    _sc1_body,
    compiler_params=_sc_params,
    out_type=[
        jax.ShapeDtypeStruct((HEADS, N_PAD, D_HEAD), jnp.float32),
        jax.ShapeDtypeStruct((HEADS, N_PAD), jnp.float32),
    ],
    mesh=_mesh,
    scratch_types=[
        pltpu.VMEM((N_PAD,), jnp.float32),
        pltpu.VMEM((N_PAD,), jnp.float32),
        pltpu.VMEM((CT1,), jnp.int32),
        pltpu.VMEM((CT1,), jnp.int32),
        [pltpu.VMEM((CHUNK,), jnp.int32)] * NBUF1,
        [pltpu.VMEM((CHUNK,), jnp.int32)] * NBUF1,
        [pltpu.VMEM((CHUNK,), jnp.float32)] * NBUF1,
        [pltpu.VMEM((CHUNK, D_HEAD), jnp.float32)] * NBUF1,
        pltpu.VMEM((ZR, D_HEAD), jnp.float32),
        pltpu.VMEM((ROWS_T,), jnp.float32),
        pltpu.VMEM((ROWS_T,), jnp.float32),
        [pltpu.SemaphoreType.DMA] * NBUF1,
        [pltpu.SemaphoreType.DMA] * NBUF1,
        [pltpu.SemaphoreType.DMA] * NBUF1,
        pltpu.VMEM_SHARED((N_PAD, D_HEAD), jnp.float32),
        pltpu.VMEM_SHARED((N_PAD,), jnp.float32),
    ],
)


def _sc2_body(g, asdt2, srcp, dstp,
              num2, den2,
              asv, adv, srcv, dstv,
              idxs, idxa, wb, rows, zrows, zden, dvbuf,
              sg, ss, sd,
              accs, dens):
    c = lax.axis_index("c")
    s = lax.axis_index("s")

    base0 = c * (16 * CT2) + s * CT2
    pltpu.sync_copy(srcp.at[pl.ds(base0, CT2)], srcv)
    pltpu.sync_copy(dstp.at[pl.ds(base0, CT2)], dstv)
    pltpu.sync_copy(asdt2.at[0], asv)
    pltpu.sync_copy(asdt2.at[1], adv)

    _zero_rows(zrows, ZR, F2 // 16)
    _zero_ref(zden, ROWS_T // 16)
    pltpu.sync_copy(zden, dens.at[pl.ds(s * ROWS_T, ROWS_T)])
    for r in range(ROWS_T // ZR):
        pltpu.sync_copy(zrows, accs.at[pl.ds(s * ROWS_T + r * ZR, ZR)])
    plsc.subcore_barrier()

    def _main(j, _):
        for b in range(NBUF2):
            k = j * NBUF2 + b

            @pl.when(j > 0)
            def _w():
                pltpu.make_async_copy(rows[b], accs.at[idxa[b]],
                                      ss[b]).wait()
                pltpu.make_async_copy(wb[b], dens.at[idxa[b]],
                                      sd[b]).wait()
            for gi in range(CHUNK // 16):
                co = gi * 16
                sl = pl.ds(k * CHUNK + co, 16)
                s16 = srcv[sl]
                d16 = dstv[sl]
                a16 = plsc.load_gather(asv, [s16])
                b16 = plsc.load_gather(adv, [d16])
                wb[b][pl.ds(co, 16)] = jnp.exp(_lrelu(a16 + b16))
                idxs[b][pl.ds(co, 16)] = s16
                idxa[b][pl.ds(co, 16)] = d16
            pltpu.async_copy(g.at[idxs[b]], rows[b], sg[b])
        for b in range(NBUF2):
            pltpu.make_async_copy(g.at[idxs[b]], rows[b], sg[b]).wait()

            def _scale(e, _):
                for u in range(EU):
                    ei = e * EU + u
                    wv = plsc.load_gather(
                        wb[b], [jnp.full((16,), ei, jnp.int32)])
                    for q in range(F2 // 16):
                        rows[b][ei, pl.ds(q * 16, 16)] = (
                            rows[b][ei, pl.ds(q * 16, 16)] * wv)
                return 0
            lax.fori_loop(0, CHUNK // EU, _scale, 0)

            pltpu.async_copy(rows[b], accs.at[idxa[b]], ss[b], add=True)
            pltpu.async_copy(wb[b], dens.at[idxa[b]], sd[b], add=True)
        return 0
    lax.fori_loop(0, NCH2 // NBUF2, _main, 0)

    for b in range(NBUF2):
        pltpu.make_async_copy(rows[b], accs.at[idxa[b]], ss[b]).wait()
        pltpu.make_async_copy(wb[b], dens.at[idxa[b]], sd[b]).wait()
    plsc.subcore_barrier()

    for r in range(ROWS_T // CHUNK):
        off = s * ROWS_T + r * CHUNK
        pltpu.sync_copy(accs.at[pl.ds(off, CHUNK)], rows[0])
        pltpu.sync_copy(rows[0], num2.at[c, pl.ds(off, CHUNK)])
    pltpu.sync_copy(dens.at[pl.ds(s * ROWS_T, ROWS_T)], dvbuf)
    pltpu.sync_copy(dvbuf, den2.at[c, pl.ds(s * ROWS_T, ROWS_T)])


_sc2 = pl.kernel(
    _sc2_body,
    compiler_params=_sc_params,
    out_type=[
        jax.ShapeDtypeStruct((2, N_PAD, F2), jnp.float32),
        jax.ShapeDtypeStruct((2, N_PAD), jnp.float32),
    ],
    mesh=_mesh,
    scratch_types=[
        pltpu.VMEM((N_PAD,), jnp.float32),
        pltpu.VMEM((N_PAD,), jnp.float32),
        pltpu.VMEM((CT2,), jnp.int32),
        pltpu.VMEM((CT2,), jnp.int32),
        [pltpu.VMEM((CHUNK,), jnp.int32)] * NBUF2,
        [pltpu.VMEM((CHUNK,), jnp.int32)] * NBUF2,
        [pltpu.VMEM((CHUNK,), jnp.float32)] * NBUF2,
        [pltpu.VMEM((CHUNK, F2), jnp.float32)] * NBUF2,
        pltpu.VMEM((ZR, F2), jnp.float32),
        pltpu.VMEM((ROWS_T,), jnp.float32),
        pltpu.VMEM((ROWS_T,), jnp.float32),
        [pltpu.SemaphoreType.DMA] * NBUF2,
        [pltpu.SemaphoreType.DMA] * NBUF2,
        [pltpu.SemaphoreType.DMA] * NBUF2,
        pltpu.VMEM_SHARED((N_PAD, F2), jnp.float32),
        pltpu.VMEM_SHARED((N_PAD,), jnp.float32),
    ],
)


def kernel(x, edge_index, W1, a_src1, a_dst1, b1, W2, a_src2, a_dst2, b2):
    loops = jnp.arange(N_NODES, dtype=edge_index.dtype)
    src = jnp.concatenate([edge_index[0], loops]).astype(jnp.int32)
    dst = jnp.concatenate([edge_index[1], loops]).astype(jnp.int32)
    srcp = jnp.concatenate(
        [src, jnp.zeros((E_PAD - E_TOT,), jnp.int32)])
    dstp = jnp.concatenate(
        [dst, jnp.full((E_PAD - E_TOT,), N_NODES, jnp.int32)])

    xp = jnp.concatenate(
        [x, jnp.zeros((N_PAD - N_NODES, D_IN), jnp.float32)], axis=0)

    eye = jnp.eye(HEADS, dtype=jnp.float32)
    A_s = (a_src1[:, :, None] * eye[:, None, :]).reshape(HEADS * D_HEAD, HEADS)
    A_d = (a_dst1[:, :, None] * eye[:, None, :]).reshape(HEADS * D_HEAD, HEADS)
    A1 = jnp.concatenate([A_s, A_d], axis=1)

    h, asd = _k1(xp, W1, A1)
    feat = h.reshape(N_PAD * HEADS, D_HEAD)
    asdt = asd.T

    num1, den1 = _sc1(feat, asdt, srcp, dstp)

    W2p = jnp.concatenate(
        [W2, jnp.zeros((HEADS * D_HEAD, F2 - N_CLASSES), jnp.float32)], axis=1)
    a2 = jnp.zeros((F2, 16), jnp.float32)
    a2 = a2.at[:N_CLASSES, 0].set(a_src2[0])
    a2 = a2.at[:N_CLASSES, 1].set(a_dst2[0])

    g, asd2 = _k2(num1, den1.T, b1.reshape(1, -1), W2p, a2)
    asdt2 = asd2.T

    num2, den2 = _sc2(g, asdt2, srcp, dstp)

    b2p = jnp.concatenate(
        [b2, jnp.zeros((F2 - N_CLASSES,), jnp.float32)]).reshape(1, F2)
    outp = _k3(num2, den2.T, b2p)
    return outp[:N_NODES, :N_CLASSES]


# speedup vs baseline: 1.0441x; 1.0328x over previous
import jax
import jax.numpy as jnp
from jax import lax
from jax.experimental import pallas as pl
from jax.experimental.pallas import tpu as pltpu, tpu_sc as plsc

N_NODES = 10000
D_IN = 256
HEADS = 8
D_HEAD = 64
N_CLASSES = 40
NEG = 0.2

N_PAD = 10240
ROWS_T = N_PAD // 16
E_TOT = N_NODES + 160000
E_PAD = 172032
CHUNK = 128
CT1 = E_PAD // 16
NCH1 = CT1 // CHUNK
CT2 = E_PAD // 32
ER1 = 10000
LP1 = 625
NCH2 = CT2 // CHUNK
NBUF1 = 4
NBUF2 = 3
F2 = 48
TILE_N = 256
GRID_N = N_PAD // TILE_N
EU = 4
ZR = 64

_mesh = plsc.VectorSubcoreMesh(core_axis_name="c", subcore_axis_name="s")


def _lrelu(x):
    return jnp.maximum(x, NEG * x)


def _k1_body(x_ref, w_ref, a_ref, h_ref, asd_ref):
    h = jnp.dot(x_ref[...], w_ref[...], preferred_element_type=jnp.float32)
    h_ref[...] = h
    asd_ref[...] = jnp.dot(h, a_ref[...], preferred_element_type=jnp.float32)


_k1 = pl.pallas_call(
    _k1_body,
    grid=(GRID_N,),
    in_specs=[
        pl.BlockSpec((TILE_N, D_IN), lambda i: (i, 0)),
        pl.BlockSpec((D_IN, HEADS * D_HEAD), lambda i: (0, 0)),
        pl.BlockSpec((HEADS * D_HEAD, 16), lambda i: (0, 0)),
    ],
    out_specs=[
        pl.BlockSpec((TILE_N, HEADS * D_HEAD), lambda i: (i, 0)),
        pl.BlockSpec((TILE_N, 16), lambda i: (i, 0)),
    ],
    out_shape=[
        jax.ShapeDtypeStruct((N_PAD, HEADS * D_HEAD), jnp.float32),
        jax.ShapeDtypeStruct((N_PAD, 16), jnp.float32),
    ],
)


def _k2_body(num_ref, den_ref, b1_ref, w2_ref, a2_ref, g_ref, asd2_ref):
    parts = []
    for hh in range(HEADS):
        t = num_ref[hh] / den_ref[:, hh:hh + 1]
        t = t + b1_ref[0, hh * D_HEAD:(hh + 1) * D_HEAD]
        parts.append(t)
    t = jnp.concatenate(parts, axis=1)
    t = jnp.where(t > 0, t, jnp.exp(t) - 1.0)
    g = jnp.dot(t, w2_ref[...], preferred_element_type=jnp.float32)
    g_ref[...] = g
    asd2_ref[...] = jnp.dot(g, a2_ref[...], preferred_element_type=jnp.float32)


_k2 = pl.pallas_call(
    _k2_body,
    grid=(GRID_N,),
    in_specs=[
        pl.BlockSpec((HEADS, TILE_N, D_HEAD), lambda i: (0, i, 0)),
        pl.BlockSpec((TILE_N, HEADS), lambda i: (i, 0)),
        pl.BlockSpec((1, HEADS * D_HEAD), lambda i: (0, 0)),
        pl.BlockSpec((HEADS * D_HEAD, F2), lambda i: (0, 0)),
        pl.BlockSpec((F2, 16), lambda i: (0, 0)),
    ],
    out_specs=[
        pl.BlockSpec((TILE_N, F2), lambda i: (i, 0)),
        pl.BlockSpec((TILE_N, 16), lambda i: (i, 0)),
    ],
    out_shape=[
        jax.ShapeDtypeStruct((N_PAD, F2), jnp.float32),
        jax.ShapeDtypeStruct((N_PAD, 16), jnp.float32),
    ],
)


def _k3_body(num_ref, den_ref, b2_ref, out_ref):
    n = num_ref[0] + num_ref[1]
    dd = den_ref[:, 0:1] + den_ref[:, 1:2]
    out_ref[...] = n / dd + b2_ref[0]


_k3 = pl.pallas_call(
    _k3_body,
    grid=(GRID_N,),
    in_specs=[
        pl.BlockSpec((2, TILE_N, F2), lambda i: (0, i, 0)),
        pl.BlockSpec((TILE_N, 2), lambda i: (i, 0)),
        pl.BlockSpec((1, F2), lambda i: (0, 0)),
    ],
    out_specs=pl.BlockSpec((TILE_N, F2), lambda i: (i, 0)),
    out_shape=jax.ShapeDtypeStruct((N_PAD, F2), jnp.float32),
)


_sc_params = pltpu.CompilerParams(needs_layout_passes=False,
                                  use_tc_tiling_on_sc=False)


def _zero_ref(ref, n16):
    def _z(i, _):
        ref[pl.ds(i * 16, 16)] = jnp.zeros((16,), jnp.float32)
        return 0
    lax.fori_loop(0, n16, _z, 0)


def _zero_rows(ref, nrows, ncol16):
    def _z(i, _):
        for q in range(ncol16):
            ref[i, pl.ds(q * 16, 16)] = jnp.zeros((16,), jnp.float32)
        return 0
    lax.fori_loop(0, nrows, _z, 0)


def _sc1_body(feat, asdt, eidx,
              num1, den1,
              asv, adv, srcv, dstv,
              idxf, idxa, wb, rows, zrows, zden, dvbuf,
              sg, ss, sd,
              accs, dens):
    c = lax.axis_index("c")
    s = lax.axis_index("s")

    pltpu.sync_copy(eidx.at[0, pl.ds(s * ER1, ER1)], srcv.at[pl.ds(0, ER1)])
    pltpu.sync_copy(eidx.at[1, pl.ds(s * ER1, ER1)], dstv.at[pl.ds(0, ER1)])

    def _gen(gi, _):
        lidx = gi * 16 + jax.lax.iota(jnp.int32, 16)
        is_loop = lidx < LP1
        lv = s * LP1 + lidx
        srcv[pl.ds(ER1 + gi * 16, 16)] = jnp.where(is_loop, lv, 0)
        dstv[pl.ds(ER1 + gi * 16, 16)] = jnp.where(is_loop, lv, N_NODES)
        return 0
    lax.fori_loop(0, (CT1 - ER1) // 16, _gen, 0)

    _zero_rows(zrows, ZR, D_HEAD // 16)
    _zero_ref(zden, ROWS_T // 16)
    pltpu.sync_copy(zden, dens.at[pl.ds(s * ROWS_T, ROWS_T)])

    for i in range(HEADS // 2):
        hg = c * (HEADS // 2) + i

        pltpu.sync_copy(asdt.at[hg], asv)
        pltpu.sync_copy(asdt.at[HEADS + hg], adv)
        for r in range(ROWS_T // ZR):
            pltpu.sync_copy(zrows, accs.at[pl.ds(s * ROWS_T + r * ZR, ZR)])
        plsc.subcore_barrier()

        def _main(j, _):
            for b in range(NBUF1):
                k = j * NBUF1 + b

                @pl.when(j > 0)
                def _w():
                    pltpu.make_async_copy(rows[b], accs.at[idxa[b]],
                                          ss[b]).wait()
                    pltpu.make_async_copy(wb[b], dens.at[idxa[b]],
                                          sd[b]).wait()
                for g in range(CHUNK // 16):
                    co = g * 16
                    sl = pl.ds(k * CHUNK + co, 16)
                    s16 = srcv[sl]
                    d16 = dstv[sl]
                    a16 = plsc.load_gather(asv, [s16])
                    b16 = plsc.load_gather(adv, [d16])
                    wb[b][pl.ds(co, 16)] = jnp.exp(_lrelu(a16 + b16))
                    idxf[b][pl.ds(co, 16)] = s16 * HEADS + hg
                    idxa[b][pl.ds(co, 16)] = d16
                pltpu.async_copy(feat.at[idxf[b]], rows[b], sg[b])
            for b in range(NBUF1):
                pltpu.make_async_copy(feat.at[idxf[b]], rows[b],
                                      sg[b]).wait()

                def _scale(e, _):
                    for u in range(EU):
                        ei = e * EU + u
                        wv = plsc.load_gather(
                            wb[b], [jnp.full((16,), ei, jnp.int32)])
                        for q in range(D_HEAD // 16):
                            rows[b][ei, pl.ds(q * 16, 16)] = (
                                rows[b][ei, pl.ds(q * 16, 16)] * wv)
                    return 0
                lax.fori_loop(0, CHUNK // EU, _scale, 0)

                pltpu.async_copy(rows[b], accs.at[idxa[b]], ss[b],
                                 add=True)
                pltpu.async_copy(wb[b], dens.at[idxa[b]], sd[b], add=True)
            return 0
        lax.fori_loop(0, NCH1 // NBUF1, _main, 0)

        for b in range(NBUF1):
            pltpu.make_async_copy(rows[b], accs.at[idxa[b]], ss[b]).wait()
            pltpu.make_async_copy(wb[b], dens.at[idxa[b]], sd[b]).wait()
        plsc.subcore_barrier()

        for r in range(ROWS_T // CHUNK):
            off = s * ROWS_T + r * CHUNK
            pltpu.sync_copy(accs.at[pl.ds(off, CHUNK)], rows[0])
            pltpu.sync_copy(rows[0], num1.at[hg, pl.ds(off, CHUNK)])
        pltpu.sync_copy(dens.at[pl.ds(s * ROWS_T, ROWS_T)], dvbuf)
        pltpu.sync_copy(dvbuf, den1.at[hg, pl.ds(s * ROWS_T, ROWS_T)])
        if i < HEADS // 2 - 1:
            pltpu.sync_copy(zden, dens.at[pl.ds(s * ROWS_T, ROWS_T)])
        plsc.subcore_barrier()


_sc1 = pl.kernel(
    _sc1_body,
    compiler_params=_sc_params,
    out_type=[
        jax.ShapeDtypeStruct((HEADS, N_PAD, D_HEAD), jnp.float32),
        jax.ShapeDtypeStruct((HEADS, N_PAD), jnp.float32),
    ],
    mesh=_mesh,
    scratch_types=[
        pltpu.VMEM((N_PAD,), jnp.float32),
        pltpu.VMEM((N_PAD,), jnp.float32),
        pltpu.VMEM((CT1,), jnp.int32),
        pltpu.VMEM((CT1,), jnp.int32),
        [pltpu.VMEM((CHUNK,), jnp.int32)] * NBUF1,
        [pltpu.VMEM((CHUNK,), jnp.int32)] * NBUF1,
        [pltpu.VMEM((CHUNK,), jnp.float32)] * NBUF1,
        [pltpu.VMEM((CHUNK, D_HEAD), jnp.float32)] * NBUF1,
        pltpu.VMEM((ZR, D_HEAD), jnp.float32),
        pltpu.VMEM((ROWS_T,), jnp.float32),
        pltpu.VMEM((ROWS_T,), jnp.float32),
        [pltpu.SemaphoreType.DMA] * NBUF1,
        [pltpu.SemaphoreType.DMA] * NBUF1,
        [pltpu.SemaphoreType.DMA] * NBUF1,
        pltpu.VMEM_SHARED((N_PAD, D_HEAD), jnp.float32),
        pltpu.VMEM_SHARED((N_PAD,), jnp.float32),
    ],
)


def _sc2_body(g, asdt2, eidx,
              num2, den2,
              asv, adv, srcv, dstv,
              idxs, idxa, wb, rows, zrows, zden, dvbuf,
              sg, ss, sd,
              accs, dens):
    c = lax.axis_index("c")
    s = lax.axis_index("s")

    @pl.when(c == 0)
    def _stage0():
        pltpu.sync_copy(eidx.at[0, pl.ds(s * ER1, CT2)], srcv)
        pltpu.sync_copy(eidx.at[1, pl.ds(s * ER1, CT2)], dstv)

    @pl.when(c == 1)
    def _stage1():
        pltpu.sync_copy(eidx.at[0, pl.ds(s * ER1 + CT2, ER1 - CT2)],
                        srcv.at[pl.ds(0, ER1 - CT2)])
        pltpu.sync_copy(eidx.at[1, pl.ds(s * ER1 + CT2, ER1 - CT2)],
                        dstv.at[pl.ds(0, ER1 - CT2)])

        def _gen(gi, _):
            lidx = ER1 - CT2 + gi * 16 + jax.lax.iota(jnp.int32, 16)
            is_loop = lidx < ER1 + LP1 - CT2
            lv = s * LP1 + (lidx - (ER1 - CT2))
            srcv[pl.ds(ER1 - CT2 + gi * 16, 16)] = jnp.where(is_loop, lv, 0)
            dstv[pl.ds(ER1 - CT2 + gi * 16, 16)] = (
                jnp.where(is_loop, lv, N_NODES))
            return 0
        lax.fori_loop(0, (CT1 - ER1) // 16, _gen, 0)
    pltpu.sync_copy(asdt2.at[0], asv)
    pltpu.sync_copy(asdt2.at[1], adv)

    _zero_rows(zrows, ZR, F2 // 16)
    _zero_ref(zden, ROWS_T // 16)
    pltpu.sync_copy(zden, dens.at[pl.ds(s * ROWS_T, ROWS_T)])
    for r in range(ROWS_T // ZR):
        pltpu.sync_copy(zrows, accs.at[pl.ds(s * ROWS_T + r * ZR, ZR)])
    plsc.subcore_barrier()

    def _main(j, _):
        for b in range(NBUF2):
            k = j * NBUF2 + b

            @pl.when(j > 0)
            def _w():
                pltpu.make_async_copy(rows[b], accs.at[idxa[b]],
                                      ss[b]).wait()
                pltpu.make_async_copy(wb[b], dens.at[idxa[b]],
                                      sd[b]).wait()
            for gi in range(CHUNK // 16):
                co = gi * 16
                sl = pl.ds(k * CHUNK + co, 16)
                s16 = srcv[sl]
                d16 = dstv[sl]
                a16 = plsc.load_gather(asv, [s16])
                b16 = plsc.load_gather(adv, [d16])
                wb[b][pl.ds(co, 16)] = jnp.exp(_lrelu(a16 + b16))
                idxs[b][pl.ds(co, 16)] = s16
                idxa[b][pl.ds(co, 16)] = d16
            pltpu.async_copy(g.at[idxs[b]], rows[b], sg[b])
        for b in range(NBUF2):
            pltpu.make_async_copy(g.at[idxs[b]], rows[b], sg[b]).wait()

            def _scale(e, _):
                for u in range(EU):
                    ei = e * EU + u
                    wv = plsc.load_gather(
                        wb[b], [jnp.full((16,), ei, jnp.int32)])
                    for q in range(F2 // 16):
                        rows[b][ei, pl.ds(q * 16, 16)] = (
                            rows[b][ei, pl.ds(q * 16, 16)] * wv)
                return 0
            lax.fori_loop(0, CHUNK // EU, _scale, 0)

            pltpu.async_copy(rows[b], accs.at[idxa[b]], ss[b], add=True)
            pltpu.async_copy(wb[b], dens.at[idxa[b]], sd[b], add=True)
        return 0
    lax.fori_loop(0, NCH2 // NBUF2, _main, 0)

    for b in range(NBUF2):
        pltpu.make_async_copy(rows[b], accs.at[idxa[b]], ss[b]).wait()
        pltpu.make_async_copy(wb[b], dens.at[idxa[b]], sd[b]).wait()
    plsc.subcore_barrier()

    for r in range(ROWS_T // CHUNK):
        off = s * ROWS_T + r * CHUNK
        pltpu.sync_copy(accs.at[pl.ds(off, CHUNK)], rows[0])
        pltpu.sync_copy(rows[0], num2.at[c, pl.ds(off, CHUNK)])
    pltpu.sync_copy(dens.at[pl.ds(s * ROWS_T, ROWS_T)], dvbuf)
    pltpu.sync_copy(dvbuf, den2.at[c, pl.ds(s * ROWS_T, ROWS_T)])


_sc2 = pl.kernel(
    _sc2_body,
    compiler_params=_sc_params,
    out_type=[
        jax.ShapeDtypeStruct((2, N_PAD, F2), jnp.float32),
        jax.ShapeDtypeStruct((2, N_PAD), jnp.float32),
    ],
    mesh=_mesh,
    scratch_types=[
        pltpu.VMEM((N_PAD,), jnp.float32),
        pltpu.VMEM((N_PAD,), jnp.float32),
        pltpu.VMEM((CT2,), jnp.int32),
        pltpu.VMEM((CT2,), jnp.int32),
        [pltpu.VMEM((CHUNK,), jnp.int32)] * NBUF2,
        [pltpu.VMEM((CHUNK,), jnp.int32)] * NBUF2,
        [pltpu.VMEM((CHUNK,), jnp.float32)] * NBUF2,
        [pltpu.VMEM((CHUNK, F2), jnp.float32)] * NBUF2,
        pltpu.VMEM((ZR, F2), jnp.float32),
        pltpu.VMEM((ROWS_T,), jnp.float32),
        pltpu.VMEM((ROWS_T,), jnp.float32),
        [pltpu.SemaphoreType.DMA] * NBUF2,
        [pltpu.SemaphoreType.DMA] * NBUF2,
        [pltpu.SemaphoreType.DMA] * NBUF2,
        pltpu.VMEM_SHARED((N_PAD, F2), jnp.float32),
        pltpu.VMEM_SHARED((N_PAD,), jnp.float32),
    ],
)


def kernel(x, edge_index, W1, a_src1, a_dst1, b1, W2, a_src2, a_dst2, b2):
    eidx = edge_index.astype(jnp.int32)

    xp = jnp.concatenate(
        [x, jnp.zeros((N_PAD - N_NODES, D_IN), jnp.float32)], axis=0)

    eye = jnp.eye(HEADS, dtype=jnp.float32)
    A_s = (a_src1[:, :, None] * eye[:, None, :]).reshape(HEADS * D_HEAD, HEADS)
    A_d = (a_dst1[:, :, None] * eye[:, None, :]).reshape(HEADS * D_HEAD, HEADS)
    A1 = jnp.concatenate([A_s, A_d], axis=1)

    h, asd = _k1(xp, W1, A1)
    feat = h.reshape(N_PAD * HEADS, D_HEAD)
    asdt = asd.T

    num1, den1 = _sc1(feat, asdt, eidx)

    W2p = jnp.concatenate(
        [W2, jnp.zeros((HEADS * D_HEAD, F2 - N_CLASSES), jnp.float32)], axis=1)
    a2 = jnp.zeros((F2, 16), jnp.float32)
    a2 = a2.at[:N_CLASSES, 0].set(a_src2[0])
    a2 = a2.at[:N_CLASSES, 1].set(a_dst2[0])

    g, asd2 = _k2(num1, den1.T, b1.reshape(1, -1), W2p, a2)
    asdt2 = asd2.T

    num2, den2 = _sc2(g, asdt2, eidx)

    b2p = jnp.concatenate(
        [b2, jnp.zeros((F2 - N_CLASSES,), jnp.float32)]).reshape(1, F2)
    outp = _k3(num2, den2.T, b2p)
    return outp[:N_NODES, :N_CLASSES]


# speedup vs baseline: 1.0511x; 1.0067x over previous
import jax
import jax.numpy as jnp
from jax import lax
from jax.experimental import pallas as pl
from jax.experimental.pallas import tpu as pltpu, tpu_sc as plsc

N_NODES = 10000
D_IN = 256
HEADS = 8
D_HEAD = 64
N_CLASSES = 40
NEG = 0.2

N_PAD = 10240
ROWS_T = N_PAD // 16
E_TOT = N_NODES + 160000
E_PAD = 172032
CHUNK = 128
CT1 = E_PAD // 16
NCH1 = CT1 // CHUNK
CT2 = E_PAD // 32
ER1 = 10000
LP1 = 625
NCH2 = CT2 // CHUNK
NBUF1 = 4
NBUF2 = 3
F2 = 48
TILE_N = 256
GRID_N = N_PAD // TILE_N
EU = 4
ZR = 64

_mesh = plsc.VectorSubcoreMesh(core_axis_name="c", subcore_axis_name="s")


def _lrelu(x):
    return jnp.maximum(x, NEG * x)


def _k1_body(x_ref, w_ref, a_ref, h_ref, asd_ref):
    h = jnp.dot(x_ref[...], w_ref[...], preferred_element_type=jnp.float32)
    h_ref[...] = h
    asd_ref[...] = jnp.dot(h, a_ref[...], preferred_element_type=jnp.float32)


_k1 = pl.pallas_call(
    _k1_body,
    grid=(GRID_N,),
    in_specs=[
        pl.BlockSpec((TILE_N, D_IN), lambda i: (i, 0)),
        pl.BlockSpec((D_IN, HEADS * D_HEAD), lambda i: (0, 0)),
        pl.BlockSpec((HEADS * D_HEAD, 16), lambda i: (0, 0)),
    ],
    out_specs=[
        pl.BlockSpec((TILE_N, HEADS * D_HEAD), lambda i: (i, 0)),
        pl.BlockSpec((TILE_N, 16), lambda i: (i, 0)),
    ],
    out_shape=[
        jax.ShapeDtypeStruct((N_PAD, HEADS * D_HEAD), jnp.float32),
        jax.ShapeDtypeStruct((N_PAD, 16), jnp.float32),
    ],
)


def _k2_body(num_ref, den_ref, b1_ref, w2_ref, a2_ref, g_ref, asd2_ref):
    parts = []
    for hh in range(HEADS):
        t = num_ref[hh] / den_ref[:, hh:hh + 1]
        t = t + b1_ref[0, hh * D_HEAD:(hh + 1) * D_HEAD]
        parts.append(t)
    t = jnp.concatenate(parts, axis=1)
    t = jnp.where(t > 0, t, jnp.exp(t) - 1.0)
    g = jnp.dot(t, w2_ref[...], preferred_element_type=jnp.float32)
    g_ref[...] = g
    asd2_ref[...] = jnp.dot(g, a2_ref[...], preferred_element_type=jnp.float32)


_k2 = pl.pallas_call(
    _k2_body,
    grid=(GRID_N,),
    in_specs=[
        pl.BlockSpec((HEADS, TILE_N, D_HEAD), lambda i: (0, i, 0)),
        pl.BlockSpec((TILE_N, HEADS), lambda i: (i, 0)),
        pl.BlockSpec((1, HEADS * D_HEAD), lambda i: (0, 0)),
        pl.BlockSpec((HEADS * D_HEAD, F2), lambda i: (0, 0)),
        pl.BlockSpec((F2, 16), lambda i: (0, 0)),
    ],
    out_specs=[
        pl.BlockSpec((TILE_N, F2), lambda i: (i, 0)),
        pl.BlockSpec((TILE_N, 16), lambda i: (i, 0)),
    ],
    out_shape=[
        jax.ShapeDtypeStruct((N_PAD, F2), jnp.float32),
        jax.ShapeDtypeStruct((N_PAD, 16), jnp.float32),
    ],
)


def _k3_body(num_ref, den_ref, b2_ref, out_ref):
    n = num_ref[0] + num_ref[1]
    dd = den_ref[:, 0:1] + den_ref[:, 1:2]
    out_ref[...] = n / dd + b2_ref[0]


_k3 = pl.pallas_call(
    _k3_body,
    grid=(GRID_N,),
    in_specs=[
        pl.BlockSpec((2, TILE_N, F2), lambda i: (0, i, 0)),
        pl.BlockSpec((TILE_N, 2), lambda i: (i, 0)),
        pl.BlockSpec((1, F2), lambda i: (0, 0)),
    ],
    out_specs=pl.BlockSpec((TILE_N, F2), lambda i: (i, 0)),
    out_shape=jax.ShapeDtypeStruct((N_PAD, F2), jnp.float32),
)


_sc_params = pltpu.CompilerParams(needs_layout_passes=False,
                                  use_tc_tiling_on_sc=False)


def _zero_ref(ref, n16):
    def _z(i, _):
        ref[pl.ds(i * 16, 16)] = jnp.zeros((16,), jnp.float32)
        return 0
    lax.fori_loop(0, n16, _z, 0)


def _zero_rows(ref, nrows, ncol16):
    def _z(i, _):
        for q in range(ncol16):
            ref[i, pl.ds(q * 16, 16)] = jnp.zeros((16,), jnp.float32)
        return 0
    lax.fori_loop(0, nrows, _z, 0)


def _sc1_body(feat, asdt, eidx,
              num1, den1,
              asv, adv, srcv, dstv,
              idxf, idxa, wb, rows, zrows, zden, dvbuf,
              sg, ss, sd,
              accs, dens):
    c = lax.axis_index("c")
    s = lax.axis_index("s")

    pltpu.sync_copy(eidx.at[0, pl.ds(s * ER1, ER1)], srcv.at[pl.ds(0, ER1)])
    pltpu.sync_copy(eidx.at[1, pl.ds(s * ER1, ER1)], dstv.at[pl.ds(0, ER1)])

    def _gen(gi, _):
        lidx = gi * 16 + jax.lax.iota(jnp.int32, 16)
        is_loop = lidx < LP1
        lv = s * LP1 + lidx
        srcv[pl.ds(ER1 + gi * 16, 16)] = jnp.where(is_loop, lv, 0)
        dstv[pl.ds(ER1 + gi * 16, 16)] = jnp.where(is_loop, lv, N_NODES)
        return 0
    lax.fori_loop(0, (CT1 - ER1) // 16, _gen, 0)

    _zero_rows(zrows, ZR, D_HEAD // 16)
    _zero_ref(zden, ROWS_T // 16)
    pltpu.sync_copy(zden, dens.at[pl.ds(s * ROWS_T, ROWS_T)])

    for i in range(HEADS // 2):
        hg = c * (HEADS // 2) + i

        pltpu.sync_copy(asdt.at[hg], asv)
        pltpu.sync_copy(asdt.at[HEADS + hg], adv)
        for r in range(ROWS_T // ZR):
            pltpu.async_copy(zrows, accs.at[pl.ds(s * ROWS_T + r * ZR, ZR)],
                             sg[r % NBUF1])
        for r in range(ROWS_T // ZR):
            pltpu.make_async_copy(zrows,
                                  accs.at[pl.ds(s * ROWS_T + r * ZR, ZR)],
                                  sg[r % NBUF1]).wait()
        plsc.subcore_barrier()

        def _main(j, _):
            for b in range(NBUF1):
                k = j * NBUF1 + b

                @pl.when(j > 0)
                def _w():
                    pltpu.make_async_copy(rows[b], accs.at[idxa[b]],
                                          ss[b]).wait()
                    pltpu.make_async_copy(wb[b], dens.at[idxa[b]],
                                          sd[b]).wait()
                for g in range(CHUNK // 16):
                    co = g * 16
                    sl = pl.ds(k * CHUNK + co, 16)
                    s16 = srcv[sl]
                    d16 = dstv[sl]
                    a16 = plsc.load_gather(asv, [s16])
                    b16 = plsc.load_gather(adv, [d16])
                    wb[b][pl.ds(co, 16)] = jnp.exp(_lrelu(a16 + b16))
                    idxf[b][pl.ds(co, 16)] = s16 * HEADS + hg
                    idxa[b][pl.ds(co, 16)] = d16
                pltpu.async_copy(feat.at[idxf[b]], rows[b], sg[b])
            for b in range(NBUF1):
                pltpu.make_async_copy(feat.at[idxf[b]], rows[b],
                                      sg[b]).wait()

                def _scale(e, _):
                    for u in range(EU):
                        ei = e * EU + u
                        wv = plsc.load_gather(
                            wb[b], [jnp.full((16,), ei, jnp.int32)])
                        for q in range(D_HEAD // 16):
                            rows[b][ei, pl.ds(q * 16, 16)] = (
                                rows[b][ei, pl.ds(q * 16, 16)] * wv)
                    return 0
                lax.fori_loop(0, CHUNK // EU, _scale, 0)

                pltpu.async_copy(rows[b], accs.at[idxa[b]], ss[b],
                                 add=True)
                pltpu.async_copy(wb[b], dens.at[idxa[b]], sd[b], add=True)
            return 0
        lax.fori_loop(0, NCH1 // NBUF1, _main, 0)

        for b in range(NBUF1):
            pltpu.make_async_copy(rows[b], accs.at[idxa[b]], ss[b]).wait()
            pltpu.make_async_copy(wb[b], dens.at[idxa[b]], sd[b]).wait()
        plsc.subcore_barrier()

        for r in range(ROWS_T // CHUNK):
            off = s * ROWS_T + r * CHUNK
            b = r % 2
            if r >= 2:
                offp = s * ROWS_T + (r - 2) * CHUNK
                pltpu.make_async_copy(rows[b],
                                      num1.at[hg, pl.ds(offp, CHUNK)],
                                      ss[b]).wait()
            pltpu.sync_copy(accs.at[pl.ds(off, CHUNK)], rows[b])
            pltpu.async_copy(rows[b], num1.at[hg, pl.ds(off, CHUNK)], ss[b])
        for r in range(ROWS_T // CHUNK - 2, ROWS_T // CHUNK):
            off = s * ROWS_T + r * CHUNK
            pltpu.make_async_copy(rows[r % 2],
                                  num1.at[hg, pl.ds(off, CHUNK)],
                                  ss[r % 2]).wait()
        pltpu.sync_copy(dens.at[pl.ds(s * ROWS_T, ROWS_T)], dvbuf)
        pltpu.sync_copy(dvbuf, den1.at[hg, pl.ds(s * ROWS_T, ROWS_T)])
        if i < HEADS // 2 - 1:
            pltpu.sync_copy(zden, dens.at[pl.ds(s * ROWS_T, ROWS_T)])
        plsc.subcore_barrier()


_sc1 = pl.kernel(
    _sc1_body,
    compiler_params=_sc_params,
    out_type=[
        jax.ShapeDtypeStruct((HEADS, N_PAD, D_HEAD), jnp.float32),
        jax.ShapeDtypeStruct((HEADS, N_PAD), jnp.float32),
    ],
    mesh=_mesh,
    scratch_types=[
        pltpu.VMEM((N_PAD,), jnp.float32),
        pltpu.VMEM((N_PAD,), jnp.float32),
        pltpu.VMEM((CT1,), jnp.int32),
        pltpu.VMEM((CT1,), jnp.int32),
        [pltpu.VMEM((CHUNK,), jnp.int32)] * NBUF1,
        [pltpu.VMEM((CHUNK,), jnp.int32)] * NBUF1,
        [pltpu.VMEM((CHUNK,), jnp.float32)] * NBUF1,
        [pltpu.VMEM((CHUNK, D_HEAD), jnp.float32)] * NBUF1,
        pltpu.VMEM((ZR, D_HEAD), jnp.float32),
        pltpu.VMEM((ROWS_T,), jnp.float32),
        pltpu.VMEM((ROWS_T,), jnp.float32),
        [pltpu.SemaphoreType.DMA] * NBUF1,
        [pltpu.SemaphoreType.DMA] * NBUF1,
        [pltpu.SemaphoreType.DMA] * NBUF1,
        pltpu.VMEM_SHARED((N_PAD, D_HEAD), jnp.float32),
        pltpu.VMEM_SHARED((N_PAD,), jnp.float32),
    ],
)


def _sc2_body(g, asdt2, eidx,
              num2, den2,
              asv, adv, srcv, dstv,
              idxs, idxa, wb, rows, zrows, zden, dvbuf,
              sg, ss, sd,
              accs, dens):
    c = lax.axis_index("c")
    s = lax.axis_index("s")

    @pl.when(c == 0)
    def _stage0():
        pltpu.sync_copy(eidx.at[0, pl.ds(s * ER1, CT2)], srcv)
        pltpu.sync_copy(eidx.at[1, pl.ds(s * ER1, CT2)], dstv)

    @pl.when(c == 1)
    def _stage1():
        pltpu.sync_copy(eidx.at[0, pl.ds(s * ER1 + CT2, ER1 - CT2)],
                        srcv.at[pl.ds(0, ER1 - CT2)])
        pltpu.sync_copy(eidx.at[1, pl.ds(s * ER1 + CT2, ER1 - CT2)],
                        dstv.at[pl.ds(0, ER1 - CT2)])

        def _gen(gi, _):
            lidx = ER1 - CT2 + gi * 16 + jax.lax.iota(jnp.int32, 16)
            is_loop = lidx < ER1 + LP1 - CT2
            lv = s * LP1 + (lidx - (ER1 - CT2))
            srcv[pl.ds(ER1 - CT2 + gi * 16, 16)] = jnp.where(is_loop, lv, 0)
            dstv[pl.ds(ER1 - CT2 + gi * 16, 16)] = (
                jnp.where(is_loop, lv, N_NODES))
            return 0
        lax.fori_loop(0, (CT1 - ER1) // 16, _gen, 0)
    pltpu.sync_copy(asdt2.at[0], asv)
    pltpu.sync_copy(asdt2.at[1], adv)

    _zero_rows(zrows, ZR, F2 // 16)
    _zero_ref(zden, ROWS_T // 16)
    pltpu.sync_copy(zden, dens.at[pl.ds(s * ROWS_T, ROWS_T)])
    for r in range(ROWS_T // ZR):
        pltpu.async_copy(zrows, accs.at[pl.ds(s * ROWS_T + r * ZR, ZR)],
                         sg[r % NBUF2])
    for r in range(ROWS_T // ZR):
        pltpu.make_async_copy(zrows,
                              accs.at[pl.ds(s * ROWS_T + r * ZR, ZR)],
                              sg[r % NBUF2]).wait()
    plsc.subcore_barrier()

    def _main(j, _):
        for b in range(NBUF2):
            k = j * NBUF2 + b

            @pl.when(j > 0)
            def _w():
                pltpu.make_async_copy(rows[b], accs.at[idxa[b]],
                                      ss[b]).wait()
                pltpu.make_async_copy(wb[b], dens.at[idxa[b]],
                                      sd[b]).wait()
            for gi in range(CHUNK // 16):
                co = gi * 16
                sl = pl.ds(k * CHUNK + co, 16)
                s16 = srcv[sl]
                d16 = dstv[sl]
                a16 = plsc.load_gather(asv, [s16])
                b16 = plsc.load_gather(adv, [d16])
                wb[b][pl.ds(co, 16)] = jnp.exp(_lrelu(a16 + b16))
                idxs[b][pl.ds(co, 16)] = s16
                idxa[b][pl.ds(co, 16)] = d16
            pltpu.async_copy(g.at[idxs[b]], rows[b], sg[b])
        for b in range(NBUF2):
            pltpu.make_async_copy(g.at[idxs[b]], rows[b], sg[b]).wait()

            def _scale(e, _):
                for u in range(EU):
                    ei = e * EU + u
                    wv = plsc.load_gather(
                        wb[b], [jnp.full((16,), ei, jnp.int32)])
                    for q in range(F2 // 16):
                        rows[b][ei, pl.ds(q * 16, 16)] = (
                            rows[b][ei, pl.ds(q * 16, 16)] * wv)
                return 0
            lax.fori_loop(0, CHUNK // EU, _scale, 0)

            pltpu.async_copy(rows[b], accs.at[idxa[b]], ss[b], add=True)
            pltpu.async_copy(wb[b], dens.at[idxa[b]], sd[b], add=True)
        return 0
    lax.fori_loop(0, NCH2 // NBUF2, _main, 0)

    for b in range(NBUF2):
        pltpu.make_async_copy(rows[b], accs.at[idxa[b]], ss[b]).wait()
        pltpu.make_async_copy(wb[b], dens.at[idxa[b]], sd[b]).wait()
    plsc.subcore_barrier()

    for r in range(ROWS_T // CHUNK):
        off = s * ROWS_T + r * CHUNK
        b = r % 2
        if r >= 2:
            offp = s * ROWS_T + (r - 2) * CHUNK
            pltpu.make_async_copy(rows[b], num2.at[c, pl.ds(offp, CHUNK)],
                                  ss[b]).wait()
        pltpu.sync_copy(accs.at[pl.ds(off, CHUNK)], rows[b])
        pltpu.async_copy(rows[b], num2.at[c, pl.ds(off, CHUNK)], ss[b])
    for r in range(ROWS_T // CHUNK - 2, ROWS_T // CHUNK):
        off = s * ROWS_T + r * CHUNK
        pltpu.make_async_copy(rows[r % 2], num2.at[c, pl.ds(off, CHUNK)],
                              ss[r % 2]).wait()
    pltpu.sync_copy(dens.at[pl.ds(s * ROWS_T, ROWS_T)], dvbuf)
    pltpu.sync_copy(dvbuf, den2.at[c, pl.ds(s * ROWS_T, ROWS_T)])


_sc2 = pl.kernel(
    _sc2_body,
    compiler_params=_sc_params,
    out_type=[
        jax.ShapeDtypeStruct((2, N_PAD, F2), jnp.float32),
        jax.ShapeDtypeStruct((2, N_PAD), jnp.float32),
    ],
    mesh=_mesh,
    scratch_types=[
        pltpu.VMEM((N_PAD,), jnp.float32),
        pltpu.VMEM((N_PAD,), jnp.float32),
        pltpu.VMEM((CT2,), jnp.int32),
        pltpu.VMEM((CT2,), jnp.int32),
        [pltpu.VMEM((CHUNK,), jnp.int32)] * NBUF2,
        [pltpu.VMEM((CHUNK,), jnp.int32)] * NBUF2,
        [pltpu.VMEM((CHUNK,), jnp.float32)] * NBUF2,
        [pltpu.VMEM((CHUNK, F2), jnp.float32)] * NBUF2,
        pltpu.VMEM((ZR, F2), jnp.float32),
        pltpu.VMEM((ROWS_T,), jnp.float32),
        pltpu.VMEM((ROWS_T,), jnp.float32),
        [pltpu.SemaphoreType.DMA] * NBUF2,
        [pltpu.SemaphoreType.DMA] * NBUF2,
        [pltpu.SemaphoreType.DMA] * NBUF2,
        pltpu.VMEM_SHARED((N_PAD, F2), jnp.float32),
        pltpu.VMEM_SHARED((N_PAD,), jnp.float32),
    ],
)


def kernel(x, edge_index, W1, a_src1, a_dst1, b1, W2, a_src2, a_dst2, b2):
    eidx = edge_index.astype(jnp.int32)

    xp = jnp.concatenate(
        [x, jnp.zeros((N_PAD - N_NODES, D_IN), jnp.float32)], axis=0)

    eye = jnp.eye(HEADS, dtype=jnp.float32)
    A_s = (a_src1[:, :, None] * eye[:, None, :]).reshape(HEADS * D_HEAD, HEADS)
    A_d = (a_dst1[:, :, None] * eye[:, None, :]).reshape(HEADS * D_HEAD, HEADS)
    A1 = jnp.concatenate([A_s, A_d], axis=1)

    h, asd = _k1(xp, W1, A1)
    feat = h.reshape(N_PAD * HEADS, D_HEAD)
    asdt = asd.T

    num1, den1 = _sc1(feat, asdt, eidx)

    W2p = jnp.concatenate(
        [W2, jnp.zeros((HEADS * D_HEAD, F2 - N_CLASSES), jnp.float32)], axis=1)
    a2 = jnp.zeros((F2, 16), jnp.float32)
    a2 = a2.at[:N_CLASSES, 0].set(a_src2[0])
    a2 = a2.at[:N_CLASSES, 1].set(a_dst2[0])

    g, asd2 = _k2(num1, den1.T, b1.reshape(1, -1), W2p, a2)
    asdt2 = asd2.T

    num2, den2 = _sc2(g, asdt2, eidx)

    b2p = jnp.concatenate(
        [b2, jnp.zeros((F2 - N_CLASSES,), jnp.float32)]).reshape(1, F2)
    outp = _k3(num2, den2.T, b2p)
    return outp[:N_NODES, :N_CLASSES]


# speedup vs baseline: 1.1817x; 1.1242x over previous
import jax
import jax.numpy as jnp
from jax import lax
from jax.experimental import pallas as pl
from jax.experimental.pallas import tpu as pltpu, tpu_sc as plsc

N_NODES = 10000
D_IN = 256
HEADS = 8
D_HEAD = 64
N_CLASSES = 40
NEG = 0.2

N_PAD = 10240
ROWS_T = N_PAD // 16
E_TOT = N_NODES + 160000
E_PAD = 172032
CHUNK = 128
CT1 = E_PAD // 16
NCH1 = CT1 // CHUNK
CT2 = E_PAD // 32
ER1 = 10000
LP1 = 625
NCH2 = CT2 // CHUNK
NBUF1 = 4
NBUF2 = 3
F2 = 48
TILE_N = 256
GRID_N = N_PAD // TILE_N
EU = 4
ZR = 64

_mesh = plsc.VectorSubcoreMesh(core_axis_name="c", subcore_axis_name="s")


def _lrelu(x):
    return jnp.maximum(x, NEG * x)


def _k1_body(x_ref, w_ref, a_ref, h_ref, asd_ref):
    h = jnp.dot(x_ref[...], w_ref[...], preferred_element_type=jnp.float32)
    h_ref[...] = h
    asd_ref[...] = jnp.dot(h, a_ref[...], preferred_element_type=jnp.float32)


_k1 = pl.pallas_call(
    _k1_body,
    grid=(GRID_N,),
    in_specs=[
        pl.BlockSpec((TILE_N, D_IN), lambda i: (i, 0)),
        pl.BlockSpec((D_IN, HEADS * D_HEAD), lambda i: (0, 0)),
        pl.BlockSpec((HEADS * D_HEAD, 16), lambda i: (0, 0)),
    ],
    out_specs=[
        pl.BlockSpec((TILE_N, HEADS * D_HEAD), lambda i: (i, 0)),
        pl.BlockSpec((TILE_N, 16), lambda i: (i, 0)),
    ],
    out_shape=[
        jax.ShapeDtypeStruct((N_PAD, HEADS * D_HEAD), jnp.float32),
        jax.ShapeDtypeStruct((N_PAD, 16), jnp.float32),
    ],
)


def _k2_body(num_ref, den_ref, b1_ref, w2_ref, a2_ref, g_ref, asd2_ref):
    parts = []
    for hh in range(HEADS):
        t = num_ref[hh] / den_ref[:, hh:hh + 1]
        t = t + b1_ref[0, hh * D_HEAD:(hh + 1) * D_HEAD]
        parts.append(t)
    t = jnp.concatenate(parts, axis=1)
    t = jnp.where(t > 0, t, jnp.exp(t) - 1.0)
    g = jnp.dot(t, w2_ref[...], preferred_element_type=jnp.float32)
    g_ref[...] = g
    asd2_ref[...] = jnp.dot(g, a2_ref[...], preferred_element_type=jnp.float32)


_k2 = pl.pallas_call(
    _k2_body,
    grid=(GRID_N,),
    in_specs=[
        pl.BlockSpec((HEADS, TILE_N, D_HEAD), lambda i: (0, i, 0)),
        pl.BlockSpec((TILE_N, HEADS), lambda i: (i, 0)),
        pl.BlockSpec((1, HEADS * D_HEAD), lambda i: (0, 0)),
        pl.BlockSpec((HEADS * D_HEAD, F2), lambda i: (0, 0)),
        pl.BlockSpec((F2, 16), lambda i: (0, 0)),
    ],
    out_specs=[
        pl.BlockSpec((TILE_N, F2), lambda i: (i, 0)),
        pl.BlockSpec((TILE_N, 16), lambda i: (i, 0)),
    ],
    out_shape=[
        jax.ShapeDtypeStruct((N_PAD, F2), jnp.float32),
        jax.ShapeDtypeStruct((N_PAD, 16), jnp.float32),
    ],
)


def _k3_body(num_ref, den_ref, b2_ref, out_ref):
    n = num_ref[0] + num_ref[1]
    dd = den_ref[:, 0:1] + den_ref[:, 1:2]
    out_ref[...] = n / dd + b2_ref[0]


_k3 = pl.pallas_call(
    _k3_body,
    grid=(GRID_N,),
    in_specs=[
        pl.BlockSpec((2, TILE_N, F2), lambda i: (0, i, 0)),
        pl.BlockSpec((TILE_N, 2), lambda i: (i, 0)),
        pl.BlockSpec((1, F2), lambda i: (0, 0)),
    ],
    out_specs=pl.BlockSpec((TILE_N, F2), lambda i: (i, 0)),
    out_shape=jax.ShapeDtypeStruct((N_PAD, F2), jnp.float32),
)


_sc_params = pltpu.CompilerParams(needs_layout_passes=False,
                                  use_tc_tiling_on_sc=False)


def _zero_ref(ref, n16):
    def _z(i, _):
        ref[pl.ds(i * 16, 16)] = jnp.zeros((16,), jnp.float32)
        return 0
    lax.fori_loop(0, n16, _z, 0)


def _zero_rows(ref, nrows, ncol16):
    def _z(i, _):
        for q in range(ncol16):
            ref[i, pl.ds(q * 16, 16)] = jnp.zeros((16,), jnp.float32)
        return 0
    lax.fori_loop(0, nrows, _z, 0)


def _sc1_body(feat, asdt, eidx,
              num1, den1,
              asv, adv, srcv, dstv,
              idxf, idxa, wb, rows, zrows, zden, dvbuf,
              sg, ss, sd,
              accs, dens):
    c = lax.axis_index("c")
    s = lax.axis_index("s")

    pltpu.sync_copy(eidx.at[0, pl.ds(s * ER1, ER1)], srcv.at[pl.ds(0, ER1)])
    pltpu.sync_copy(eidx.at[1, pl.ds(s * ER1, ER1)], dstv.at[pl.ds(0, ER1)])

    def _gen(gi, _):
        lidx = gi * 16 + jax.lax.iota(jnp.int32, 16)
        is_loop = lidx < LP1
        lv = s * LP1 + lidx
        srcv[pl.ds(ER1 + gi * 16, 16)] = jnp.where(is_loop, lv, 0)
        dstv[pl.ds(ER1 + gi * 16, 16)] = jnp.where(is_loop, lv, N_NODES)
        return 0
    lax.fori_loop(0, (CT1 - ER1) // 16, _gen, 0)

    _zero_rows(zrows, ZR, D_HEAD // 16)
    _zero_ref(zden, ROWS_T // 16)
    pltpu.sync_copy(zden, dens.at[pl.ds(s * ROWS_T, ROWS_T)])

    for i in range(HEADS // 2):
        hg = c * (HEADS // 2) + i

        pltpu.sync_copy(asdt.at[hg], asv)
        pltpu.sync_copy(asdt.at[HEADS + hg], adv)
        for r in range(ROWS_T // ZR):
            pltpu.async_copy(zrows, accs.at[pl.ds(s * ROWS_T + r * ZR, ZR)],
                             sg[r % NBUF1])
        for r in range(ROWS_T // ZR):
            pltpu.make_async_copy(zrows,
                                  accs.at[pl.ds(s * ROWS_T + r * ZR, ZR)],
                                  sg[r % NBUF1]).wait()
        plsc.subcore_barrier()

        def _main(j, _):
            for b in range(NBUF1):
                k = j * NBUF1 + b

                @pl.when(j > 0)
                def _w():
                    pltpu.make_async_copy(rows[b], accs.at[idxa[b]],
                                          ss[b]).wait()
                    pltpu.make_async_copy(wb[b], dens.at[idxa[b]],
                                          sd[b]).wait()
                for g in range(CHUNK // 16):
                    co = g * 16
                    sl = pl.ds(k * CHUNK + co, 16)
                    s16 = srcv[sl]
                    d16 = dstv[sl]
                    a16 = plsc.load_gather(asv, [s16])
                    b16 = plsc.load_gather(adv, [d16])
                    wb[b][pl.ds(co, 16)] = jnp.exp(_lrelu(a16 + b16))
                    idxf[b][pl.ds(co, 16)] = s16 * HEADS + hg
                    idxa[b][pl.ds(co, 16)] = d16
                pltpu.async_copy(feat.at[idxf[b]], rows[b], sg[b])
            for b in range(NBUF1):
                pltpu.make_async_copy(feat.at[idxf[b]], rows[b],
                                      sg[b]).wait()

                def _scale(e, _):
                    w16 = wb[b][pl.ds(e * 16, 16)]
                    for u in range(16):
                        ei = e * 16 + u
                        wv = jax.lax.gather(
                            w16, jnp.full((16, 1), u, jnp.int32),
                            jax.lax.GatherDimensionNumbers(
                                offset_dims=(), collapsed_slice_dims=(0,),
                                start_index_map=(0,)),
                            (1,), mode=jax.lax.GatherScatterMode.PROMISE_IN_BOUNDS)
                        for q in range(D_HEAD // 16):
                            rows[b][ei, pl.ds(q * 16, 16)] = (
                                rows[b][ei, pl.ds(q * 16, 16)] * wv)
                    return 0
                lax.fori_loop(0, CHUNK // 16, _scale, 0)

                pltpu.async_copy(rows[b], accs.at[idxa[b]], ss[b],
                                 add=True)
                pltpu.async_copy(wb[b], dens.at[idxa[b]], sd[b], add=True)
            return 0
        lax.fori_loop(0, NCH1 // NBUF1, _main, 0)

        for b in range(NBUF1):
            pltpu.make_async_copy(rows[b], accs.at[idxa[b]], ss[b]).wait()
            pltpu.make_async_copy(wb[b], dens.at[idxa[b]], sd[b]).wait()
        plsc.subcore_barrier()

        for r in range(ROWS_T // CHUNK):
            off = s * ROWS_T + r * CHUNK
            b = r % 2
            if r >= 2:
                offp = s * ROWS_T + (r - 2) * CHUNK
                pltpu.make_async_copy(rows[b],
                                      num1.at[hg, pl.ds(offp, CHUNK)],
                                      ss[b]).wait()
            pltpu.sync_copy(accs.at[pl.ds(off, CHUNK)], rows[b])
            pltpu.async_copy(rows[b], num1.at[hg, pl.ds(off, CHUNK)], ss[b])
        for r in range(ROWS_T // CHUNK - 2, ROWS_T // CHUNK):
            off = s * ROWS_T + r * CHUNK
            pltpu.make_async_copy(rows[r % 2],
                                  num1.at[hg, pl.ds(off, CHUNK)],
                                  ss[r % 2]).wait()
        pltpu.sync_copy(dens.at[pl.ds(s * ROWS_T, ROWS_T)], dvbuf)
        pltpu.sync_copy(dvbuf, den1.at[hg, pl.ds(s * ROWS_T, ROWS_T)])
        if i < HEADS // 2 - 1:
            pltpu.sync_copy(zden, dens.at[pl.ds(s * ROWS_T, ROWS_T)])
        plsc.subcore_barrier()


_sc1 = pl.kernel(
    _sc1_body,
    compiler_params=_sc_params,
    out_type=[
        jax.ShapeDtypeStruct((HEADS, N_PAD, D_HEAD), jnp.float32),
        jax.ShapeDtypeStruct((HEADS, N_PAD), jnp.float32),
    ],
    mesh=_mesh,
    scratch_types=[
        pltpu.VMEM((N_PAD,), jnp.float32),
        pltpu.VMEM((N_PAD,), jnp.float32),
        pltpu.VMEM((CT1,), jnp.int32),
        pltpu.VMEM((CT1,), jnp.int32),
        [pltpu.VMEM((CHUNK,), jnp.int32)] * NBUF1,
        [pltpu.VMEM((CHUNK,), jnp.int32)] * NBUF1,
        [pltpu.VMEM((CHUNK,), jnp.float32)] * NBUF1,
        [pltpu.VMEM((CHUNK, D_HEAD), jnp.float32)] * NBUF1,
        pltpu.VMEM((ZR, D_HEAD), jnp.float32),
        pltpu.VMEM((ROWS_T,), jnp.float32),
        pltpu.VMEM((ROWS_T,), jnp.float32),
        [pltpu.SemaphoreType.DMA] * NBUF1,
        [pltpu.SemaphoreType.DMA] * NBUF1,
        [pltpu.SemaphoreType.DMA] * NBUF1,
        pltpu.VMEM_SHARED((N_PAD, D_HEAD), jnp.float32),
        pltpu.VMEM_SHARED((N_PAD,), jnp.float32),
    ],
)


def _sc2_body(g, asdt2, eidx,
              num2, den2,
              asv, adv, srcv, dstv,
              idxs, idxa, wb, rows, zrows, zden, dvbuf,
              sg, ss, sd,
              accs, dens):
    c = lax.axis_index("c")
    s = lax.axis_index("s")

    @pl.when(c == 0)
    def _stage0():
        pltpu.sync_copy(eidx.at[0, pl.ds(s * ER1, CT2)], srcv)
        pltpu.sync_copy(eidx.at[1, pl.ds(s * ER1, CT2)], dstv)

    @pl.when(c == 1)
    def _stage1():
        pltpu.sync_copy(eidx.at[0, pl.ds(s * ER1 + CT2, ER1 - CT2)],
                        srcv.at[pl.ds(0, ER1 - CT2)])
        pltpu.sync_copy(eidx.at[1, pl.ds(s * ER1 + CT2, ER1 - CT2)],
                        dstv.at[pl.ds(0, ER1 - CT2)])

        def _gen(gi, _):
            lidx = ER1 - CT2 + gi * 16 + jax.lax.iota(jnp.int32, 16)
            is_loop = lidx < ER1 + LP1 - CT2
            lv = s * LP1 + (lidx - (ER1 - CT2))
            srcv[pl.ds(ER1 - CT2 + gi * 16, 16)] = jnp.where(is_loop, lv, 0)
            dstv[pl.ds(ER1 - CT2 + gi * 16, 16)] = (
                jnp.where(is_loop, lv, N_NODES))
            return 0
        lax.fori_loop(0, (CT1 - ER1) // 16, _gen, 0)
    pltpu.sync_copy(asdt2.at[0], asv)
    pltpu.sync_copy(asdt2.at[1], adv)

    _zero_rows(zrows, ZR, F2 // 16)
    _zero_ref(zden, ROWS_T // 16)
    pltpu.sync_copy(zden, dens.at[pl.ds(s * ROWS_T, ROWS_T)])
    for r in range(ROWS_T // ZR):
        pltpu.async_copy(zrows, accs.at[pl.ds(s * ROWS_T + r * ZR, ZR)],
                         sg[r % NBUF2])
    for r in range(ROWS_T // ZR):
        pltpu.make_async_copy(zrows,
                              accs.at[pl.ds(s * ROWS_T + r * ZR, ZR)],
                              sg[r % NBUF2]).wait()
    plsc.subcore_barrier()

    def _main(j, _):
        for b in range(NBUF2):
            k = j * NBUF2 + b

            @pl.when(j > 0)
            def _w():
                pltpu.make_async_copy(rows[b], accs.at[idxa[b]],
                                      ss[b]).wait()
                pltpu.make_async_copy(wb[b], dens.at[idxa[b]],
                                      sd[b]).wait()
            for gi in range(CHUNK // 16):
                co = gi * 16
                sl = pl.ds(k * CHUNK + co, 16)
                s16 = srcv[sl]
                d16 = dstv[sl]
                a16 = plsc.load_gather(asv, [s16])
                b16 = plsc.load_gather(adv, [d16])
                wb[b][pl.ds(co, 16)] = jnp.exp(_lrelu(a16 + b16))
                idxs[b][pl.ds(co, 16)] = s16
                idxa[b][pl.ds(co, 16)] = d16
            pltpu.async_copy(g.at[idxs[b]], rows[b], sg[b])
        for b in range(NBUF2):
            pltpu.make_async_copy(g.at[idxs[b]], rows[b], sg[b]).wait()

            def _scale(e, _):
                w16 = wb[b][pl.ds(e * 16, 16)]
                for u in range(16):
                    ei = e * 16 + u
                    wv = jax.lax.gather(
                        w16, jnp.full((16, 1), u, jnp.int32),
                        jax.lax.GatherDimensionNumbers(
                            offset_dims=(), collapsed_slice_dims=(0,),
                            start_index_map=(0,)),
                        (1,), mode=jax.lax.GatherScatterMode.PROMISE_IN_BOUNDS)
                    for q in range(F2 // 16):
                        rows[b][ei, pl.ds(q * 16, 16)] = (
                            rows[b][ei, pl.ds(q * 16, 16)] * wv)
                return 0
            lax.fori_loop(0, CHUNK // 16, _scale, 0)

            pltpu.async_copy(rows[b], accs.at[idxa[b]], ss[b], add=True)
            pltpu.async_copy(wb[b], dens.at[idxa[b]], sd[b], add=True)
        return 0
    lax.fori_loop(0, NCH2 // NBUF2, _main, 0)

    for b in range(NBUF2):
        pltpu.make_async_copy(rows[b], accs.at[idxa[b]], ss[b]).wait()
        pltpu.make_async_copy(wb[b], dens.at[idxa[b]], sd[b]).wait()
    plsc.subcore_barrier()

    for r in range(ROWS_T // CHUNK):
        off = s * ROWS_T + r * CHUNK
        b = r % 2
        if r >= 2:
            offp = s * ROWS_T + (r - 2) * CHUNK
            pltpu.make_async_copy(rows[b], num2.at[c, pl.ds(offp, CHUNK)],
                                  ss[b]).wait()
        pltpu.sync_copy(accs.at[pl.ds(off, CHUNK)], rows[b])
        pltpu.async_copy(rows[b], num2.at[c, pl.ds(off, CHUNK)], ss[b])
    for r in range(ROWS_T // CHUNK - 2, ROWS_T // CHUNK):
        off = s * ROWS_T + r * CHUNK
        pltpu.make_async_copy(rows[r % 2], num2.at[c, pl.ds(off, CHUNK)],
                              ss[r % 2]).wait()
    pltpu.sync_copy(dens.at[pl.ds(s * ROWS_T, ROWS_T)], dvbuf)
    pltpu.sync_copy(dvbuf, den2.at[c, pl.ds(s * ROWS_T, ROWS_T)])


_sc2 = pl.kernel(
    _sc2_body,
    compiler_params=_sc_params,
    out_type=[
        jax.ShapeDtypeStruct((2, N_PAD, F2), jnp.float32),
        jax.ShapeDtypeStruct((2, N_PAD), jnp.float32),
    ],
    mesh=_mesh,
    scratch_types=[
        pltpu.VMEM((N_PAD,), jnp.float32),
        pltpu.VMEM((N_PAD,), jnp.float32),
        pltpu.VMEM((CT2,), jnp.int32),
        pltpu.VMEM((CT2,), jnp.int32),
        [pltpu.VMEM((CHUNK,), jnp.int32)] * NBUF2,
        [pltpu.VMEM((CHUNK,), jnp.int32)] * NBUF2,
        [pltpu.VMEM((CHUNK,), jnp.float32)] * NBUF2,
        [pltpu.VMEM((CHUNK, F2), jnp.float32)] * NBUF2,
        pltpu.VMEM((ZR, F2), jnp.float32),
        pltpu.VMEM((ROWS_T,), jnp.float32),
        pltpu.VMEM((ROWS_T,), jnp.float32),
        [pltpu.SemaphoreType.DMA] * NBUF2,
        [pltpu.SemaphoreType.DMA] * NBUF2,
        [pltpu.SemaphoreType.DMA] * NBUF2,
        pltpu.VMEM_SHARED((N_PAD, F2), jnp.float32),
        pltpu.VMEM_SHARED((N_PAD,), jnp.float32),
    ],
)


def kernel(x, edge_index, W1, a_src1, a_dst1, b1, W2, a_src2, a_dst2, b2):
    eidx = edge_index.astype(jnp.int32)

    xp = jnp.concatenate(
        [x, jnp.zeros((N_PAD - N_NODES, D_IN), jnp.float32)], axis=0)

    eye = jnp.eye(HEADS, dtype=jnp.float32)
    A_s = (a_src1[:, :, None] * eye[:, None, :]).reshape(HEADS * D_HEAD, HEADS)
    A_d = (a_dst1[:, :, None] * eye[:, None, :]).reshape(HEADS * D_HEAD, HEADS)
    A1 = jnp.concatenate([A_s, A_d], axis=1)

    h, asd = _k1(xp, W1, A1)
    feat = h.reshape(N_PAD * HEADS, D_HEAD)
    asdt = asd.T

    num1, den1 = _sc1(feat, asdt, eidx)

    W2p = jnp.concatenate(
        [W2, jnp.zeros((HEADS * D_HEAD, F2 - N_CLASSES), jnp.float32)], axis=1)
    a2 = jnp.zeros((F2, 16), jnp.float32)
    a2 = a2.at[:N_CLASSES, 0].set(a_src2[0])
    a2 = a2.at[:N_CLASSES, 1].set(a_dst2[0])

    g, asd2 = _k2(num1, den1.T, b1.reshape(1, -1), W2p, a2)
    asdt2 = asd2.T

    num2, den2 = _sc2(g, asdt2, eidx)

    b2p = jnp.concatenate(
        [b2, jnp.zeros((F2 - N_CLASSES,), jnp.float32)]).reshape(1, F2)
    outp = _k3(num2, den2.T, b2p)
    return outp[:N_NODES, :N_CLASSES]
